# baseline (speedup 1.0000x reference)
import functools

import jax
import jax.numpy as jnp
from jax import lax
from jax.experimental import pallas as pl
from jax.experimental.pallas import tpu as pltpu
from jax.experimental.pallas import tpu_sc as plsc

_NUM_TABLE_ROWS = 100000
_DIM = 300
_DIM_PAD = 384
_G = _DIM_PAD // 128
_DT = 304 // 8

_INFO = plsc.get_sparse_core_info()
_NC = _INFO.num_cores
_NS = _INFO.num_subcores
_NW = _NC * _NS

_A = 4096
_W = 50
_AB = 64
_NCH = _W * (_A // _AB)
_CPW = _NCH // _NW
_IPC = _AB * _G


def _make_body(t):
    def _sc_body(idx_hbm, table_hbm, out, idx_v, gb, stages, gsems, wsems):
        cid = lax.axis_index("c")
        sid = lax.axis_index("s")
        wid = sid * _NC + cid
        base = (t * _NCH + wid * _CPW) * _IPC
        pltpu.sync_copy(idx_hbm.at[pl.ds(base, _CPW * _IPC)], idx_v)

        iota = lax.iota(jnp.int32, 16)
        iota_shr3 = lax.shift_right_logical(iota, 3)
        i8_vec = lax.bitwise_and(iota, 7)
        dt_vecs = [[iota_shr3 + (g * 16 + 2 * k) for k in range(8)]
                   for g in range(_G)]
        col_vecs = [iota + 16 * k for k in range(8)]
        kmax = (8, 8, 3)

        def g_start(j, b):
            for g in range(_G):
                pltpu.make_async_copy(
                    table_hbm.at[idx_v.at[pl.ds(j * _IPC + g * _AB, _AB)]],
                    gb[b * _G + g], gsems[b]).start()

        def g_wait(j, b):
            for g in range(_G):
                pltpu.make_async_copy(
                    table_hbm.at[idx_v.at[pl.ds(j * _IPC + g * _AB, _AB)]],
                    gb[b * _G + g], gsems[b]).wait()

        def w_copy(j, b):
            m = wid * _CPW + j
            w = m // _AB
            h = m % _AB
            return pltpu.make_async_copy(
                stages[b],
                out.at[w, :, h // 2, :, pl.ds((h % 2) * _AB, _AB)],
                wsems[b])

        def transpose(b):
            stage = stages[b]

            @pl.loop(0, _AB)
            def _(a):
                a_vec = jnp.full((16,), a, jnp.int32)
                for g in range(_G):
                    src = gb[b * _G + g]
                    for k in range(kmax[g]):
                        v = plsc.load_gather(src, [a_vec, col_vecs[k]])
                        plsc.store_scatter(stage, [dt_vecs[g][k], i8_vec,
                                                   a_vec], v)

        g_start(0, 0)

        @pl.loop(0, _CPW, step=2)
        def _(j):
            for b in range(2):
                jj = j + b

                @pl.when(jj + 1 < _CPW)
                def _():
                    g_start(jj + 1, (b + 1) % 2)

                g_wait(jj, b)

                @pl.when(jj >= 2)
                def _():
                    w_copy(jj - 2, b).wait()

                transpose(b)
                w_copy(jj, b).start()

        w_copy(_CPW - 2, 0).wait()
        w_copy(_CPW - 1, 1).wait()

    return _sc_body


def _make_gather(t):
    return functools.partial(
        pl.kernel,
        out_type=jax.ShapeDtypeStruct((_W, _DT, _A // 128, 8, 128),
                                      jnp.float32),
        mesh=plsc.VectorSubcoreMesh(core_axis_name="c", subcore_axis_name="s"),
        scratch_types=[
            pltpu.VMEM((_CPW * _IPC,), jnp.int32),
            tuple(pltpu.VMEM((_AB, 128), jnp.float32) for _ in range(2 * _G)),
            tuple(pltpu.VMEM((_DT, 8, _AB), jnp.float32) for _ in range(2)),
            tuple(pltpu.SemaphoreType.DMA for _ in range(2)),
            tuple(pltpu.SemaphoreType.DMA for _ in range(2)),
        ],
        compiler_params=pltpu.CompilerParams(
            use_tc_tiling_on_sc=False, needs_layout_passes=False),
    )(_make_body(t))


_gathers = [_make_gather(t) for t in range(3)]


def kernel(x, embedding_weight):
    wt = embedding_weight.T
    wtp = jnp.pad(wt, ((0, _DIM_PAD - _DIM), (0, 0)))
    table3 = wtp.reshape(_G, 128, _NUM_TABLE_ROWS).transpose(2, 0, 1)
    table3 = table3.reshape(_NUM_TABLE_ROWS * _G, 128)

    xr = x.transpose(0, 2, 1).reshape(3, _W, _A // _AB, _AB)
    idx3 = (xr[:, :, :, None, :] * _G
            + jnp.arange(_G, dtype=jnp.int32)[None, None, None, :, None])
    idx3 = idx3.reshape(-1)

    def unpad(o5):
        o = o5.transpose(2, 4, 0, 1, 3).reshape(_A, _W, 304)
        return o[:, :, :_DIM]

    return tuple(unpad(_gathers[t](idx3, table3)) for t in range(3))
